# baseline (speedup 1.0000x reference)
import functools

import jax
import jax.numpy as jnp
from jax import lax
from jax.experimental import pallas as pl
from jax.experimental.pallas import tpu as pltpu
from jax.experimental.pallas import tpu_sc as plsc

EMB_D = 64
GCHUNK = 128


@functools.cache
def _build_kernel(n_seq: int, n_batch: int, nw: int):
    b_per_w = n_batch // nw
    u_per_w = b_per_w // GCHUNK
    n_chunks = n_seq * u_per_w
    dp = EMB_D // 8
    tjn = n_batch // GCHUNK
    mesh = plsc.VectorSubcoreMesh(core_axis_name="c", subcore_axis_name="s")

    @functools.partial(
        pl.kernel,
        mesh=mesh,
        compiler_params=pltpu.CompilerParams(
            use_tc_tiling_on_sc=False, needs_layout_passes=False
        ),
        out_type=jax.ShapeDtypeStruct((n_seq, dp, tjn, 8, GCHUNK), jnp.float32),
        scratch_types=[
            pltpu.VMEM((n_chunks, GCHUNK), jnp.int32),
            pltpu.VMEM((2, GCHUNK, EMB_D), jnp.float32),
            pltpu.VMEM((2, dp, 8, GCHUNK), jnp.float32),
            pltpu.SemaphoreType.DMA,
            pltpu.SemaphoreType.DMA,
        ],
    )
    def emb(idx_hbm, table_hbm, out_hbm, idx_v, rows_v, trans_v, gsem0, gsem1):
        cid = lax.axis_index("c")
        sid = lax.axis_index("s")
        wid = sid * 2 + cid

        pltpu.sync_copy(idx_hbm.at[wid], idx_v)

        gsems = (gsem0, gsem1)
        lanes = lax.iota(jnp.int32, 16)

        def start_gather(g, b):
            pltpu.async_copy(table_hbm.at[idx_v.at[g]], rows_v.at[b], gsems[b])

        def wait_gather(b):
            pltpu.make_async_copy(
                table_hbm.at[idx_v.at[0]], rows_v.at[b], gsems[b]
            ).wait()

        def transpose_chunk(b):
            rows = rows_v.at[b]
            trans = trans_v.at[b]

            def tg_body(tg, carry):
                ridx = tg * 16 + lanes
                for d in range(EMB_D):
                    cidx = jnp.full((16,), d, jnp.int32)
                    vec = plsc.load_gather(rows, [ridx, cidx])
                    trans[d // 8, d % 8, pl.ds(tg * 16, 16)] = vec
                return carry

            lax.fori_loop(0, GCHUNK // 16, tg_body, 0)

        start_gather(0, 0)
        start_gather(1, 1)

        def body(o, carry):
            for b in range(2):
                g = o * 2 + b
                wait_gather(b)
                transpose_chunk(b)

                @pl.when(g + 2 < n_chunks)
                def _():
                    start_gather(g + 2, b)

                s = g // u_per_w
                u = g % u_per_w
                tj = wid * u_per_w + u
                pltpu.sync_copy(trans_v.at[b], out_hbm.at[s, :, tj])
            return carry

        lax.fori_loop(0, n_chunks // 2, body, 0)

    return emb


def kernel(token_ids, weight):
    n_batch, n_seq = token_ids.shape
    nw = 32
    b_per_w = n_batch // nw
    u_per_w = b_per_w // GCHUNK
    assert b_per_w * nw == n_batch and u_per_w * GCHUNK == b_per_w

    idx3 = (
        token_ids.T.astype(jnp.int32)
        .reshape(n_seq, nw, u_per_w, GCHUNK)
        .transpose(1, 0, 2, 3)
        .reshape(nw, n_seq * u_per_w, GCHUNK)
    )
    out5 = _build_kernel(n_seq, n_batch, nw)(idx3, weight)
    return out5.transpose(2, 4, 0, 1, 3).reshape(n_batch, n_seq, EMB_D)


# speedup vs baseline: 1.6995x; 1.6995x over previous
import functools

import jax
import jax.numpy as jnp
from jax import lax
from jax.experimental import pallas as pl
from jax.experimental.pallas import tpu as pltpu
from jax.experimental.pallas import tpu_sc as plsc

EMB_D = 64
GCHUNK = 128


@functools.cache
def _build_kernel(n_seq: int, n_batch: int, nw: int):
    b_per_w = n_batch // nw
    u_per_w = b_per_w // GCHUNK
    n_chunks = n_seq * u_per_w
    dp = EMB_D // 8
    tjn = n_batch // GCHUNK
    mesh = plsc.VectorSubcoreMesh(core_axis_name="c", subcore_axis_name="s")

    @functools.partial(
        pl.kernel,
        mesh=mesh,
        compiler_params=pltpu.CompilerParams(
            use_tc_tiling_on_sc=False, needs_layout_passes=False
        ),
        out_type=jax.ShapeDtypeStruct((n_seq, dp, tjn, 8, GCHUNK), jnp.float32),
        scratch_types=[
            pltpu.VMEM((n_chunks, GCHUNK), jnp.int32),
            pltpu.VMEM((2, GCHUNK, EMB_D), jnp.float32),
            pltpu.VMEM((2, EMB_D, 129), jnp.float32),
            pltpu.SemaphoreType.DMA,
            pltpu.SemaphoreType.DMA,
        ],
    )
    def emb(idx_hbm, table_hbm, out_hbm, idx_v, rows_v, trans_v, gsem0, gsem1):
        cid = lax.axis_index("c")
        sid = lax.axis_index("s")
        wid = sid * 2 + cid

        pltpu.sync_copy(idx_hbm.at[wid], idx_v)

        gsems = (gsem0, gsem1)
        lanes = lax.iota(jnp.int32, 16)

        def start_gather(g, b):
            pltpu.async_copy(table_hbm.at[idx_v.at[g]], rows_v.at[b], gsems[b])

        def wait_gather(b):
            pltpu.make_async_copy(
                table_hbm.at[idx_v.at[0]], rows_v.at[b], gsems[b]
            ).wait()

        def transpose_chunk(b):
            rows = rows_v.at[b]
            trans = trans_v.at[b]

            def t_body(tb, carry):
                for tt in range(8):
                    t = tb * 8 + tt
                    tidx = t + jnp.zeros((16,), jnp.int32)
                    for j in range(EMB_D // 16):
                        vec = rows[t, pl.ds(j * 16, 16)]
                        plsc.store_scatter(trans, [j * 16 + lanes, tidx], vec)
                return carry

            lax.fori_loop(0, GCHUNK // 8, t_body, 0)

        start_gather(0, 0)
        start_gather(1, 1)

        def body(o, carry):
            for b in range(2):
                g = o * 2 + b
                wait_gather(b)
                transpose_chunk(b)

                @pl.when(g + 2 < n_chunks)
                def _():
                    start_gather(g + 2, b)

                s = g // u_per_w
                u = g % u_per_w
                tj = wid * u_per_w + u
                for dpi in range(dp):
                    pltpu.sync_copy(
                        trans_v.at[b].at[pl.ds(dpi * 8, 8), pl.ds(0, GCHUNK)],
                        out_hbm.at[s, dpi, tj],
                    )
            return carry

        lax.fori_loop(0, n_chunks // 2, body, 0)

    return emb


def kernel(token_ids, weight):
    n_batch, n_seq = token_ids.shape
    nw = 32
    b_per_w = n_batch // nw
    u_per_w = b_per_w // GCHUNK
    assert b_per_w * nw == n_batch and u_per_w * GCHUNK == b_per_w

    idx3 = (
        token_ids.T.astype(jnp.int32)
        .reshape(n_seq, nw, u_per_w, GCHUNK)
        .transpose(1, 0, 2, 3)
        .reshape(nw, n_seq * u_per_w, GCHUNK)
    )
    out5 = _build_kernel(n_seq, n_batch, nw)(idx3, weight)
    return out5.transpose(2, 4, 0, 1, 3).reshape(n_batch, n_seq, EMB_D)


# speedup vs baseline: 1.8016x; 1.0600x over previous
import functools

import jax
import jax.numpy as jnp
from jax import lax
from jax.experimental import pallas as pl
from jax.experimental.pallas import tpu as pltpu
from jax.experimental.pallas import tpu_sc as plsc

EMB_D = 64
GCHUNK = 128


@functools.cache
def _build_kernel(n_seq: int, n_batch: int, nw: int):
    b_per_w = n_batch // nw
    u_per_w = b_per_w // GCHUNK
    n_chunks = n_seq * u_per_w
    dp = EMB_D // 8
    tjn = n_batch // GCHUNK
    mesh = plsc.VectorSubcoreMesh(core_axis_name="c", subcore_axis_name="s")

    @functools.partial(
        pl.kernel,
        mesh=mesh,
        compiler_params=pltpu.CompilerParams(
            use_tc_tiling_on_sc=False, needs_layout_passes=False
        ),
        out_type=jax.ShapeDtypeStruct((n_seq, dp, tjn, 8, GCHUNK), jnp.float32),
        scratch_types=[
            pltpu.VMEM((n_chunks, GCHUNK), jnp.int32),
            pltpu.VMEM((2, GCHUNK, 2 * EMB_D), jnp.float32),
            pltpu.VMEM((2, EMB_D, 129), jnp.float32),
            pltpu.SemaphoreType.DMA,
            pltpu.SemaphoreType.DMA,
        ],
    )
    def emb(idx_hbm, table_hbm, out_hbm, idx_v, rows_v, trans_v, gsem0, gsem1):
        cid = lax.axis_index("c")
        sid = lax.axis_index("s")
        wid = sid * 2 + cid

        pltpu.sync_copy(idx_hbm.at[wid], idx_v)

        gsems = (gsem0, gsem1)
        lanes = lax.iota(jnp.int32, 16)

        def start_gather(g, b):
            pltpu.async_copy(table_hbm.at[idx_v.at[g]], rows_v.at[b], gsems[b])

        def wait_gather(b):
            pltpu.make_async_copy(
                table_hbm.at[idx_v.at[0]], rows_v.at[b], gsems[b]
            ).wait()

        def transpose_chunk(b):
            rows = rows_v.at[b]
            trans = trans_v.at[b]

            def t_body(tb, carry):
                for tt in range(8):
                    t = tb * 8 + tt
                    tidx = t + jnp.zeros((16,), jnp.int32)
                    for j in range(EMB_D // 16):
                        vec = rows[t, pl.ds(j * 16, 16)]
                        plsc.store_scatter(trans, [j * 16 + lanes, tidx], vec)
                return carry

            lax.fori_loop(0, GCHUNK // 8, t_body, 0)

        start_gather(0, 0)
        start_gather(1, 1)

        def body(o, carry):
            for b in range(2):
                g = o * 2 + b
                wait_gather(b)
                transpose_chunk(b)

                @pl.when(g + 2 < n_chunks)
                def _():
                    start_gather(g + 2, b)

                s = g // u_per_w
                u = g % u_per_w
                tj = wid * u_per_w + u
                for dpi in range(dp):
                    pltpu.sync_copy(
                        trans_v.at[b].at[pl.ds(dpi * 8, 8), pl.ds(0, GCHUNK)],
                        out_hbm.at[s, dpi, tj],
                    )
            return carry

        lax.fori_loop(0, n_chunks // 2, body, 0)

    return emb


def kernel(token_ids, weight):
    n_batch, n_seq = token_ids.shape
    nw = 32
    b_per_w = n_batch // nw
    u_per_w = b_per_w // GCHUNK
    assert b_per_w * nw == n_batch and u_per_w * GCHUNK == b_per_w

    idx3 = (
        token_ids.T.astype(jnp.int32)
        .reshape(n_seq, nw, u_per_w, GCHUNK)
        .transpose(1, 0, 2, 3)
        .reshape(nw, n_seq * u_per_w, GCHUNK)
    )
    wpad = jnp.pad(weight, ((0, 0), (0, 2 * EMB_D - weight.shape[1])))
    out5 = _build_kernel(n_seq, n_batch, nw)(idx3, wpad)
    return out5.transpose(2, 4, 0, 1, 3).reshape(n_batch, n_seq, EMB_D)
